# baseline (speedup 1.0000x reference)
import functools

import jax
import jax.numpy as jnp
from jax import lax
from jax.experimental import pallas as pl
from jax.experimental.pallas import tpu as pltpu, tpu_sc as plsc

_NT = 16
_NB2 = 512
_NB3 = 63
_TBL2 = _NB2 * _NB2
_TBL3 = _NB3 ** 3
_NW = 32
_C = 2048
_SUB = _C // 128


def _tec_body(p_hbm, v_hbm, wp, wv, wr, off_hbm, out,
              p_v, v_v, pp_v, vp_v, rp_v, idx, gb, gb2, off_v, sem, sem2,
              *, n):
    npw = n // _NW
    nchunks = npw // _C
    wid = lax.axis_index("s") * 2 + lax.axis_index("c")

    pltpu.sync_copy(off_hbm, off_v)

    def drain(s, cnt):
        def w(j, c):
            pltpu.make_async_copy(wp.at[pl.ds(0, 128)], gb.at[0, 0], s).wait()
            return c
        lax.fori_loop(0, cnt, w, 0)

    def fire(tbl, g, s):
        def f(j, c):
            t = j >> 4
            sub = j & (_SUB - 1)
            pltpu.async_copy(tbl.at[idx.at[t, sub]], g.at[t, sub], s)
            return c
        lax.fori_loop(0, _NT * _SUB, f, 0)

    def chunk(ci, carry):
        base = pl.multiple_of(wid * npw + ci * _C, _C)
        pltpu.sync_copy(p_hbm.at[pl.ds(base, _C)], p_v)
        pltpu.sync_copy(v_hbm.at[pl.ds(base, _C)], v_v)
        off2 = off_v[0, :]
        off3 = off_v[1, :]

        def idx2(k, c):
            sub = k >> 3
            col = (k & 7) * 16
            s16 = pl.ds(k * 16, 16)
            up = p_v[s16] * jnp.float32(_NB2)
            uv = v_v[s16] * jnp.float32(_NB2)
            for t in range(_NT):
                sh = jnp.float32(t / _NT)
                ip = jnp.minimum((up + sh).astype(jnp.int32), _NB2 - 1)
                iv = jnp.minimum((uv + sh).astype(jnp.int32), _NB2 - 1)
                idx[t, sub, pl.ds(col, 16)] = ip + (iv << 9) + (off2 + t * _TBL2)
            return c
        lax.fori_loop(0, _C // 16, idx2, 0)

        fire(wp, gb, sem)
        fire(wr, gb2, sem2)
        drain(sem, _NT * _SUB)

        def red_p(sub, c):
            for j in range(8):
                sl = pl.ds(j * 16, 16)
                acc = gb[0, sub, sl]
                for t in range(1, _NT):
                    acc = acc + gb[t, sub, sl]
                ns = pl.ds(sub * 128 + j * 16, 16)
                pp_v[ns] = jnp.clip(p_v[ns] + acc, 0.0, 1.0)
            return c
        lax.fori_loop(0, _SUB, red_p, 0)

        drain(sem2, _NT * _SUB)

        def idx3(k, c):
            sub = k >> 3
            col = (k & 7) * 16
            s16 = pl.ds(k * 16, 16)
            u0 = p_v[s16] * jnp.float32(_NB3)
            u1 = v_v[s16] * jnp.float32(_NB3)
            u2 = pp_v[s16] * jnp.float32(_NB3)
            for t in range(_NT):
                sh = jnp.float32(t / _NT)
                i0 = jnp.minimum((u0 + sh).astype(jnp.int32), _NB3 - 1)
                i1 = jnp.minimum((u1 + sh).astype(jnp.int32), _NB3 - 1)
                i2 = jnp.minimum((u2 + sh).astype(jnp.int32), _NB3 - 1)
                idx[t, sub, pl.ds(col, 16)] = (
                    i0 + i1 * _NB3 + i2 * (_NB3 * _NB3) + (off3 + t * _TBL3))
            return c
        lax.fori_loop(0, _C // 16, idx3, 0)

        fire(wv, gb, sem)

        def red_r(sub, c):
            for j in range(8):
                sl = pl.ds(j * 16, 16)
                acc = gb2[0, sub, sl]
                for t in range(1, _NT):
                    acc = acc + gb2[t, sub, sl]
                rp_v[pl.ds(sub * 128 + j * 16, 16)] = acc
            return c
        lax.fori_loop(0, _SUB, red_r, 0)

        drain(sem, _NT * _SUB)

        def red_v(sub, c):
            for j in range(8):
                sl = pl.ds(j * 16, 16)
                acc = gb[0, sub, sl]
                for t in range(1, _NT):
                    acc = acc + gb[t, sub, sl]
                ns = pl.ds(sub * 128 + j * 16, 16)
                vp_v[ns] = jnp.clip(v_v[ns] + acc, 0.0, 1.0)
            return c
        lax.fori_loop(0, _SUB, red_v, 0)

        pltpu.sync_copy(pp_v, out.at[pl.ds(base, _C)])
        pltpu.sync_copy(vp_v, out.at[pl.ds(n + base, _C)])
        pltpu.sync_copy(rp_v, out.at[pl.ds(2 * n + base, _C)])
        return carry

    lax.fori_loop(0, nchunks, chunk, 0)


def kernel(state, W_p, W_v, W_r, action):
    n = state.shape[0]
    p_in = state[:, 0]
    v_in = state[:, 1]
    wp = W_p.reshape(-1)
    wv = W_v.reshape(-1)
    wr = W_r.reshape(-1)
    a = jnp.clip(jnp.asarray(action, jnp.int32), 0, W_p.shape[0] - 1)
    off = jnp.stack([
        jnp.full((16,), a * (_NT * _TBL2), dtype=jnp.int32),
        jnp.full((16,), a * (_NT * _TBL3), dtype=jnp.int32),
    ])
    mesh = plsc.VectorSubcoreMesh(core_axis_name="c", subcore_axis_name="s")
    out = pl.kernel(
        functools.partial(_tec_body, n=n),
        out_type=jax.ShapeDtypeStruct((3 * n,), jnp.float32),
        mesh=mesh,
        scratch_types=[
            pltpu.VMEM((_C,), jnp.float32),
            pltpu.VMEM((_C,), jnp.float32),
            pltpu.VMEM((_C,), jnp.float32),
            pltpu.VMEM((_C,), jnp.float32),
            pltpu.VMEM((_C,), jnp.float32),
            pltpu.VMEM((_NT, _SUB, 128), jnp.int32),
            pltpu.VMEM((_NT, _SUB, 128), jnp.float32),
            pltpu.VMEM((_NT, _SUB, 128), jnp.float32),
            pltpu.VMEM((2, 16), jnp.int32),
            pltpu.SemaphoreType.DMA,
            pltpu.SemaphoreType.DMA,
        ],
    )(p_in, v_in, wp, wv, wr, off)
    return out.reshape(3, n).T


# speedup vs baseline: 1.6168x; 1.6168x over previous
import functools

import jax
import jax.numpy as jnp
from jax import lax
from jax.experimental import pallas as pl
from jax.experimental.pallas import tpu as pltpu, tpu_sc as plsc

_NT = 16
_NB2 = 512
_NB3 = 63
_TBL2 = _NB2 * _NB2
_TBL3 = _NB3 ** 3
_TBL3P = 250112
_NW = 32
_C = 2048
_SUB = _C // 128


def _tec_body(p_hbm, v_hbm, wp, wv, wr, off_hbm, out,
              p_v, v_v, pp_v, vp_v, rp_v, idx, gb, gb2, off_v, sem, sem2,
              *, n):
    npw = n // _NW
    nchunks = npw // _C
    wid = lax.axis_index("s") * 2 + lax.axis_index("c")

    pltpu.sync_copy(off_hbm, off_v)

    def drain(s, cnt):
        def w(j, c):
            pltpu.make_async_copy(wp.at[pl.ds(0, 128)], gb.at[0, 0], s).wait()
            return c
        lax.fori_loop(0, cnt, w, 0)

    def fire(tbl, g, s):
        def f(j, c):
            t = j >> 4
            sub = j & (_SUB - 1)
            pltpu.async_copy(tbl.at[idx.at[t, sub]], g.at[t, sub], s)
            return c
        lax.fori_loop(0, _NT * _SUB, f, 0)

    def chunk(ci, carry):
        base = pl.multiple_of(wid * npw + ci * _C, _C)
        pltpu.sync_copy(p_hbm.at[pl.ds(base, _C)], p_v)
        pltpu.sync_copy(v_hbm.at[pl.ds(base, _C)], v_v)
        off2 = off_v[0, :]
        off3 = off_v[1, :]

        def idx2(k, c):
            sub = k >> 3
            col = (k & 7) * 16
            s16 = pl.ds(k * 16, 16)
            up = p_v[s16] * jnp.float32(_NB2)
            uv = v_v[s16] * jnp.float32(_NB2)
            for t in range(_NT):
                sh = jnp.float32(t / _NT)
                ip = jnp.minimum((up + sh).astype(jnp.int32), _NB2 - 1)
                iv = jnp.minimum((uv + sh).astype(jnp.int32), _NB2 - 1)
                idx[t, sub, pl.ds(col, 16)] = ip + (iv << 9) + (off2 + t * _TBL2)
            return c
        lax.fori_loop(0, _C // 16, idx2, 0)

        fire(wp, gb, sem)
        fire(wr, gb2, sem2)
        drain(sem, _NT * _SUB)

        def red_p(sub, c):
            for j in range(8):
                sl = pl.ds(j * 16, 16)
                acc = gb[0, sub, sl]
                for t in range(1, _NT):
                    acc = acc + gb[t, sub, sl]
                ns = pl.ds(sub * 128 + j * 16, 16)
                pp_v[ns] = jnp.clip(p_v[ns] + acc, 0.0, 1.0)
            return c
        lax.fori_loop(0, _SUB, red_p, 0)

        drain(sem2, _NT * _SUB)

        def idx3(k, c):
            sub = k >> 3
            col = (k & 7) * 16
            s16 = pl.ds(k * 16, 16)
            u0 = p_v[s16] * jnp.float32(_NB3)
            u1 = v_v[s16] * jnp.float32(_NB3)
            u2 = pp_v[s16] * jnp.float32(_NB3)
            for t in range(_NT):
                sh = jnp.float32(t / _NT)
                i0 = jnp.minimum((u0 + sh).astype(jnp.int32), _NB3 - 1)
                i1 = jnp.minimum((u1 + sh).astype(jnp.int32), _NB3 - 1)
                i2 = jnp.minimum((u2 + sh).astype(jnp.int32), _NB3 - 1)
                idx[t, sub, pl.ds(col, 16)] = (
                    i0 + i1 * _NB3 + i2 * (_NB3 * _NB3) + (off3 + t * _TBL3P))
            return c
        lax.fori_loop(0, _C // 16, idx3, 0)

        fire(wv, gb, sem)

        def red_r(sub, c):
            for j in range(8):
                sl = pl.ds(j * 16, 16)
                acc = gb2[0, sub, sl]
                for t in range(1, _NT):
                    acc = acc + gb2[t, sub, sl]
                rp_v[pl.ds(sub * 128 + j * 16, 16)] = acc
            return c
        lax.fori_loop(0, _SUB, red_r, 0)

        drain(sem, _NT * _SUB)

        def red_v(sub, c):
            for j in range(8):
                sl = pl.ds(j * 16, 16)
                acc = gb[0, sub, sl]
                for t in range(1, _NT):
                    acc = acc + gb[t, sub, sl]
                ns = pl.ds(sub * 128 + j * 16, 16)
                vp_v[ns] = jnp.clip(v_v[ns] + acc, 0.0, 1.0)
            return c
        lax.fori_loop(0, _SUB, red_v, 0)

        pltpu.sync_copy(pp_v, out.at[pl.ds(base, _C)])
        pltpu.sync_copy(vp_v, out.at[pl.ds(n + base, _C)])
        pltpu.sync_copy(rp_v, out.at[pl.ds(2 * n + base, _C)])
        return carry

    lax.fori_loop(0, nchunks, chunk, 0)


def kernel(state, W_p, W_v, W_r, action):
    n = state.shape[0]
    p_in = state[:, 0]
    v_in = state[:, 1]
    wp = W_p.reshape(-1)
    wv = jnp.pad(W_v, ((0, 0), (0, 0), (0, _TBL3P - W_v.shape[2]))).reshape(-1)
    wr = W_r.reshape(-1)
    a = jnp.clip(jnp.asarray(action, jnp.int32), 0, W_p.shape[0] - 1)
    off = jnp.stack([
        jnp.full((16,), a * (_NT * _TBL2), dtype=jnp.int32),
        jnp.full((16,), a * (_NT * _TBL3P), dtype=jnp.int32),
    ])
    mesh = plsc.VectorSubcoreMesh(core_axis_name="c", subcore_axis_name="s")
    out = pl.kernel(
        functools.partial(_tec_body, n=n),
        out_type=jax.ShapeDtypeStruct((3 * n,), jnp.float32),
        mesh=mesh,
        scratch_types=[
            pltpu.VMEM((_C,), jnp.float32),
            pltpu.VMEM((_C,), jnp.float32),
            pltpu.VMEM((_C,), jnp.float32),
            pltpu.VMEM((_C,), jnp.float32),
            pltpu.VMEM((_C,), jnp.float32),
            pltpu.VMEM((_NT, _SUB, 128), jnp.int32),
            pltpu.VMEM((_NT, _SUB, 128), jnp.float32),
            pltpu.VMEM((_NT, _SUB, 128), jnp.float32),
            pltpu.VMEM((2, 16), jnp.int32),
            pltpu.SemaphoreType.DMA,
            pltpu.SemaphoreType.DMA,
        ],
    )(p_in, v_in, wp, wv, wr, off)
    return out.reshape(3, n).T


# speedup vs baseline: 1.7398x; 1.0761x over previous
import functools

import jax
import jax.numpy as jnp
from jax import lax
from jax.experimental import pallas as pl
from jax.experimental.pallas import tpu as pltpu, tpu_sc as plsc

_NT = 16
_NB2 = 512
_NB3 = 63
_TBL2 = _NB2 * _NB2
_TBL3 = _NB3 ** 3
_TBL3P = 250112
_NW = 32
_C = 2048
_SUB = _C // 128


def _tec_body(p_hbm, v_hbm, wp, wv, wr, off_hbm, out,
              p_v, v_v, pp_v, vp_v, rp_v, idx, gb, gb2, off_v, sem, sem2,
              *, n):
    npw = n // _NW
    nchunks = npw // _C
    wid = lax.axis_index("s") * 2 + lax.axis_index("c")

    pltpu.sync_copy(off_hbm, off_v)

    def drain(s, cnt):
        def w(j, c):
            pltpu.make_async_copy(wp.at[pl.ds(0, 128)], gb.at[0, 0], s).wait()
            return c
        lax.fori_loop(0, cnt, w, 0)

    def fire(tbl, g, s):
        def f(j, c):
            t = j >> 4
            sub = j & (_SUB - 1)
            pltpu.async_copy(tbl.at[idx.at[t, sub]], g.at[t, sub], s)
            return c
        lax.fori_loop(0, _NT * _SUB, f, 0)

    def chunk(ci, carry):
        base = pl.multiple_of(wid * npw + ci * _C, _C)
        pltpu.sync_copy(p_hbm.at[pl.ds(base, _C)], p_v)
        pltpu.sync_copy(v_hbm.at[pl.ds(base, _C)], v_v)
        off2 = off_v[0, :]
        off3 = off_v[1, :]

        def idx2(k, c):
            sub = k >> 3
            col = (k & 7) * 16
            s16 = pl.ds(k * 16, 16)
            up = p_v[s16] * jnp.float32(_NB2)
            uv = v_v[s16] * jnp.float32(_NB2)
            for t in range(_NT):
                sh = jnp.float32(t / _NT)
                ip = jnp.minimum((up + sh).astype(jnp.int32), _NB2 - 1)
                iv = jnp.minimum((uv + sh).astype(jnp.int32), _NB2 - 1)
                tconst = (t >> 3) * (8 * _TBL2) + (t & 7) * 128
                idx[t, sub, pl.ds(col, 16)] = (
                    ((ip >> 7) << 10) + (iv << 12) + (ip & 127) + (off2 + tconst))
            return c
        lax.fori_loop(0, _C // 16, idx2, 0)

        fire(wp, gb, sem)
        fire(wr, gb2, sem2)
        drain(sem, _NT * _SUB)

        def red_p(sub, c):
            for j in range(8):
                sl = pl.ds(j * 16, 16)
                acc = gb[0, sub, sl]
                for t in range(1, _NT):
                    acc = acc + gb[t, sub, sl]
                ns = pl.ds(sub * 128 + j * 16, 16)
                pp_v[ns] = jnp.clip(p_v[ns] + acc, 0.0, 1.0)
            return c
        lax.fori_loop(0, _SUB, red_p, 0)

        drain(sem2, _NT * _SUB)

        def idx3(k, c):
            sub = k >> 3
            col = (k & 7) * 16
            s16 = pl.ds(k * 16, 16)
            u0 = p_v[s16] * jnp.float32(_NB3)
            u1 = v_v[s16] * jnp.float32(_NB3)
            u2 = pp_v[s16] * jnp.float32(_NB3)
            for t in range(_NT):
                sh = jnp.float32(t / _NT)
                i0 = jnp.minimum((u0 + sh).astype(jnp.int32), _NB3 - 1)
                i1 = jnp.minimum((u1 + sh).astype(jnp.int32), _NB3 - 1)
                i2 = jnp.minimum((u2 + sh).astype(jnp.int32), _NB3 - 1)
                idx[t, sub, pl.ds(col, 16)] = (
                    i0 + i1 * _NB3 + i2 * (_NB3 * _NB3) + (off3 + t * _TBL3P))
            return c
        lax.fori_loop(0, _C // 16, idx3, 0)

        fire(wv, gb, sem)

        def red_r(sub, c):
            for j in range(8):
                sl = pl.ds(j * 16, 16)
                acc = gb2[0, sub, sl]
                for t in range(1, _NT):
                    acc = acc + gb2[t, sub, sl]
                rp_v[pl.ds(sub * 128 + j * 16, 16)] = acc
            return c
        lax.fori_loop(0, _SUB, red_r, 0)

        drain(sem, _NT * _SUB)

        def red_v(sub, c):
            for j in range(8):
                sl = pl.ds(j * 16, 16)
                acc = gb[0, sub, sl]
                for t in range(1, _NT):
                    acc = acc + gb[t, sub, sl]
                ns = pl.ds(sub * 128 + j * 16, 16)
                vp_v[ns] = jnp.clip(v_v[ns] + acc, 0.0, 1.0)
            return c
        lax.fori_loop(0, _SUB, red_v, 0)

        pltpu.sync_copy(pp_v, out.at[pl.ds(base, _C)])
        pltpu.sync_copy(vp_v, out.at[pl.ds(n + base, _C)])
        pltpu.sync_copy(rp_v, out.at[pl.ds(2 * n + base, _C)])
        return carry

    lax.fori_loop(0, nchunks, chunk, 0)


def kernel(state, W_p, W_v, W_r, action):
    n = state.shape[0]
    p_in = state[:, 0]
    v_in = state[:, 1]
    def _phys(W):
        na, nt, nf = W.shape
        return W.reshape(na, nt // 8, 8, nf // 128, 128).transpose(0, 1, 3, 2, 4).reshape(-1)

    wp = _phys(W_p)
    wv = jnp.pad(W_v, ((0, 0), (0, 0), (0, _TBL3P - W_v.shape[2]))).reshape(-1)
    wr = _phys(W_r)
    a = jnp.clip(jnp.asarray(action, jnp.int32), 0, W_p.shape[0] - 1)
    off = jnp.stack([
        jnp.full((16,), a * (_NT * _TBL2), dtype=jnp.int32),
        jnp.full((16,), a * (_NT * _TBL3P), dtype=jnp.int32),
    ])
    mesh = plsc.VectorSubcoreMesh(core_axis_name="c", subcore_axis_name="s")
    out = pl.kernel(
        functools.partial(_tec_body, n=n),
        out_type=jax.ShapeDtypeStruct((3 * n,), jnp.float32),
        mesh=mesh,
        scratch_types=[
            pltpu.VMEM((_C,), jnp.float32),
            pltpu.VMEM((_C,), jnp.float32),
            pltpu.VMEM((_C,), jnp.float32),
            pltpu.VMEM((_C,), jnp.float32),
            pltpu.VMEM((_C,), jnp.float32),
            pltpu.VMEM((_NT, _SUB, 128), jnp.int32),
            pltpu.VMEM((_NT, _SUB, 128), jnp.float32),
            pltpu.VMEM((_NT, _SUB, 128), jnp.float32),
            pltpu.VMEM((2, 16), jnp.int32),
            pltpu.SemaphoreType.DMA,
            pltpu.SemaphoreType.DMA,
        ],
    )(p_in, v_in, wp, wv, wr, off)
    return out.reshape(3, n).T


# speedup vs baseline: 1.7539x; 1.0081x over previous
import functools

import jax
import jax.numpy as jnp
from jax import lax
from jax.experimental import pallas as pl
from jax.experimental.pallas import tpu as pltpu, tpu_sc as plsc

_NT = 16
_NB2 = 512
_NB3 = 63
_TBL2 = _NB2 * _NB2
_TBL3 = _NB3 ** 3
_TBL3P = 250112
_NW = 32
_C = 1024
_SUB = _C // 128


def _tec_body(p_hbm, v_hbm, wp, wv, wr, off_hbm, out,
              p_v, v_v, pp_v, vp_v, rp_v, idx2, idx3, gp, gr, gv, off_v,
              sem1, sem2, semv, semo, *, n):
    npw = n // _NW
    nchunks = npw // _C
    wid = lax.axis_index("s") * 2 + lax.axis_index("c")
    w0 = wid * npw

    pltpu.sync_copy(off_hbm, off_v)

    def drain(s, cnt):
        def w(j, c):
            pltpu.make_async_copy(wp.at[pl.ds(0, 128)], gp.at[0, 0], s).wait()
            return c
        lax.fori_loop(0, cnt, w, 0)

    def fire(tbl, idx, g, s):
        def f(j, c):
            t = j // _SUB
            sub = j % _SUB
            pltpu.async_copy(tbl.at[idx.at[t, sub]], g.at[t, sub], s)
            return c
        lax.fori_loop(0, _NT * _SUB, f, 0)

    def draino():
        for _ in range(3):
            pltpu.make_async_copy(p_hbm.at[pl.ds(0, _C)], pp_v.at[0], semo).wait()

    def load_idx2_fire(ci, par):
        base = pl.multiple_of(w0 + ci * _C, _C)
        pltpu.sync_copy(p_hbm.at[pl.ds(base, _C)], p_v.at[par])
        pltpu.sync_copy(v_hbm.at[pl.ds(base, _C)], v_v.at[par])
        off2 = off_v[0, :]

        def body(k, c):
            sub = k >> 3
            col = (k & 7) * 16
            s16 = pl.ds(k * 16, 16)
            up = p_v[par, s16] * jnp.float32(_NB2)
            uv = v_v[par, s16] * jnp.float32(_NB2)
            for t in range(_NT):
                sh = jnp.float32(t / _NT)
                ip = jnp.minimum((up + sh).astype(jnp.int32), _NB2 - 1)
                iv = jnp.minimum((uv + sh).astype(jnp.int32), _NB2 - 1)
                tconst = (t >> 3) * (8 * _TBL2) + (t & 7) * 128
                idx2[t, sub, pl.ds(col, 16)] = (
                    ((ip >> 7) << 10) + (iv << 12) + (ip & 127) + (off2 + tconst))
            return c
        lax.fori_loop(0, _C // 16, body, 0)
        fire(wp, idx2, gp, sem1)
        fire(wr, idx2, gr, sem2)

    def red(g, src2, dst2, do_clip):
        def body(sub, c):
            for j in range(8):
                sl = pl.ds(j * 16, 16)
                acc = g[0, sub, sl]
                for t in range(1, _NT):
                    acc = acc + g[t, sub, sl]
                ns = pl.ds(sub * 128 + j * 16, 16)
                if do_clip:
                    dst2[ns] = jnp.clip(src2[ns] + acc, 0.0, 1.0)
                else:
                    dst2[ns] = acc
            return c
        lax.fori_loop(0, _SUB, body, 0)

    def idx3_fire(ci, par, pp1):
        off3 = off_v[1, :]

        def body(k, c):
            sub = k >> 3
            col = (k & 7) * 16
            s16 = pl.ds(k * 16, 16)
            u0 = p_v[par, s16] * jnp.float32(_NB3)
            u1 = v_v[par, s16] * jnp.float32(_NB3)
            u2 = pp1[s16] * jnp.float32(_NB3)
            for t in range(_NT):
                sh = jnp.float32(t / _NT)
                i0 = jnp.minimum((u0 + sh).astype(jnp.int32), _NB3 - 1)
                i1 = jnp.minimum((u1 + sh).astype(jnp.int32), _NB3 - 1)
                i2 = jnp.minimum((u2 + sh).astype(jnp.int32), _NB3 - 1)
                idx3[t, sub, pl.ds(col, 16)] = (
                    i0 + i1 * _NB3 + i2 * (_NB3 * _NB3) + (off3 + t * _TBL3P))
            return c
        lax.fori_loop(0, _C // 16, body, 0)
        fire(wv, idx3, gv, semv)

    def finish_v(ci, par):
        base = pl.multiple_of(w0 + ci * _C, _C)
        drain(semv, _NT * _SUB)
        red(gv, v_v.at[par], vp_v.at[par], True)
        pltpu.async_copy(pp_v.at[par], out.at[pl.ds(base, _C)], semo)
        pltpu.async_copy(vp_v.at[par], out.at[pl.ds(n + base, _C)], semo)
        pltpu.async_copy(rp_v.at[par], out.at[pl.ds(2 * n + base, _C)], semo)

    load_idx2_fire(0, 0)

    def chunk(ci, carry):
        par = ci & 1

        @pl.when(ci > 1)
        def _():
            draino()

        @pl.when(ci > 0)
        def _():
            finish_v(ci - 1, 1 - par)

        drain(sem1, _NT * _SUB)
        red(gp, p_v.at[par], pp_v.at[par], True)
        drain(sem2, _NT * _SUB)
        red(gr, None, rp_v.at[par], False)
        idx3_fire(ci, par, pp_v.at[par])

        @pl.when(ci + 1 < nchunks)
        def _():
            load_idx2_fire(ci + 1, 1 - par)
        return carry

    lax.fori_loop(0, nchunks, chunk, 0)
    draino()
    finish_v(nchunks - 1, (nchunks - 1) & 1)
    draino()


def kernel(state, W_p, W_v, W_r, action):
    n = state.shape[0]
    p_in = state[:, 0]
    v_in = state[:, 1]

    def _phys(W):
        na, nt, nf = W.shape
        return W.reshape(na, nt // 8, 8, nf // 128, 128).transpose(0, 1, 3, 2, 4).reshape(-1)

    wp = _phys(W_p)
    wv = jnp.pad(W_v, ((0, 0), (0, 0), (0, _TBL3P - W_v.shape[2]))).reshape(-1)
    wr = _phys(W_r)
    a = jnp.clip(jnp.asarray(action, jnp.int32), 0, W_p.shape[0] - 1)
    off = jnp.stack([
        jnp.full((16,), a * (_NT * _TBL2), dtype=jnp.int32),
        jnp.full((16,), a * (_NT * _TBL3P), dtype=jnp.int32),
    ])
    mesh = plsc.VectorSubcoreMesh(core_axis_name="c", subcore_axis_name="s")
    out = pl.kernel(
        functools.partial(_tec_body, n=n),
        out_type=jax.ShapeDtypeStruct((3 * n,), jnp.float32),
        mesh=mesh,
        scratch_types=[
            pltpu.VMEM((2, _C), jnp.float32),
            pltpu.VMEM((2, _C), jnp.float32),
            pltpu.VMEM((2, _C), jnp.float32),
            pltpu.VMEM((2, _C), jnp.float32),
            pltpu.VMEM((2, _C), jnp.float32),
            pltpu.VMEM((_NT, _SUB, 128), jnp.int32),
            pltpu.VMEM((_NT, _SUB, 128), jnp.int32),
            pltpu.VMEM((_NT, _SUB, 128), jnp.float32),
            pltpu.VMEM((_NT, _SUB, 128), jnp.float32),
            pltpu.VMEM((_NT, _SUB, 128), jnp.float32),
            pltpu.VMEM((2, 16), jnp.int32),
            pltpu.SemaphoreType.DMA,
            pltpu.SemaphoreType.DMA,
            pltpu.SemaphoreType.DMA,
            pltpu.SemaphoreType.DMA,
        ],
    )(p_in, v_in, wp, wv, wr, off)
    return out.reshape(3, n).T


# speedup vs baseline: 1.8239x; 1.0399x over previous
import functools

import jax
import jax.numpy as jnp
from jax import lax
from jax.experimental import pallas as pl
from jax.experimental.pallas import tpu as pltpu, tpu_sc as plsc

_NT = 16
_NB2 = 512
_NB3 = 63
_TBL2 = _NB2 * _NB2
_TBL3 = _NB3 ** 3
_TBL3P = 250112
_NW = 32
_C = 1024
_SUB = _C // 128


def _wid():
    return lax.axis_index("s") * 2 + lax.axis_index("c")


def _red16(g, src2, dst2, do_clip):
    def body(sub, c):
        for j in range(8):
            sl = pl.ds(j * 16, 16)
            acc = g[0, sub, sl]
            for t in range(1, _NT):
                acc = acc + g[t, sub, sl]
            ns = pl.ds(sub * 128 + j * 16, 16)
            if do_clip:
                dst2[ns] = jnp.clip(src2[ns] + acc, 0.0, 1.0)
            else:
                dst2[ns] = acc
        return c
    lax.fori_loop(0, _SUB, body, 0)


def _k1_body(p_hbm, v_hbm, wp, wr, off_hbm, out,
             p_v, v_v, pp_v, rp_v, idx2, gp, gr, off_v, sem1, sem2, semo,
             *, n):
    npw = n // _NW
    nchunks = npw // _C
    w0 = _wid() * npw
    pltpu.sync_copy(off_hbm, off_v)

    def drain(s, cnt):
        def w(j, c):
            pltpu.make_async_copy(wp.at[pl.ds(0, 128)], gp.at[0, 0, 0], s).wait()
            return c
        lax.fori_loop(0, cnt, w, 0)

    def draino(k):
        for _ in range(k):
            pltpu.make_async_copy(p_hbm.at[pl.ds(0, _C)], pp_v.at[0], semo).wait()

    def load_idx2_fire(ci, par):
        base = pl.multiple_of(w0 + ci * _C, _C)
        pltpu.sync_copy(p_hbm.at[pl.ds(base, _C)], p_v.at[par])
        pltpu.sync_copy(v_hbm.at[pl.ds(base, _C)], v_v.at[par])
        off2 = off_v[...]

        def body(k, c):
            sub = k >> 3
            col = (k & 7) * 16
            s16 = pl.ds(k * 16, 16)
            up = p_v[par, s16] * jnp.float32(_NB2)
            uv = v_v[par, s16] * jnp.float32(_NB2)
            for t in range(_NT):
                sh = jnp.float32(t / _NT)
                ip = jnp.minimum((up + sh).astype(jnp.int32), _NB2 - 1)
                iv = jnp.minimum((uv + sh).astype(jnp.int32), _NB2 - 1)
                tconst = (t >> 3) * (8 * _TBL2) + (t & 7) * 128
                idx2[par, t, sub, pl.ds(col, 16)] = (
                    ((ip >> 7) << 10) + (iv << 12) + (ip & 127) + (off2 + tconst))
            return c
        lax.fori_loop(0, _C // 16, body, 0)

        def f(j, c):
            t = j // _SUB
            sub = j % _SUB
            pltpu.async_copy(wp.at[idx2.at[par, t, sub]], gp.at[par, t, sub], sem1)
            pltpu.async_copy(wr.at[idx2.at[par, t, sub]], gr.at[par, t, sub], sem2)
            return c
        lax.fori_loop(0, _NT * _SUB, f, 0)

    load_idx2_fire(0, 0)

    def chunk(ci, carry):
        par = ci & 1
        drain(sem1, _NT * _SUB)

        @pl.when(ci + 1 < nchunks)
        def _():
            load_idx2_fire(ci + 1, 1 - par)

        @pl.when(ci > 1)
        def _():
            draino(2)

        base = pl.multiple_of(w0 + ci * _C, _C)
        _red16(gp.at[par], p_v.at[par], pp_v.at[par], True)
        pltpu.async_copy(pp_v.at[par], out.at[pl.ds(base, _C)], semo)
        drain(sem2, _NT * _SUB)
        _red16(gr.at[par], None, rp_v.at[par], False)
        pltpu.async_copy(rp_v.at[par], out.at[pl.ds(n + base, _C)], semo)
        return carry

    lax.fori_loop(0, nchunks, chunk, 0)
    draino(4)


def _k2_body(p_hbm, v_hbm, ppr_hbm, wv, off_hbm, out,
             p_v, v_v, pp_v, vp_v, idx3, gv, off_v, semv, semo, *, n):
    npw = n // _NW
    nchunks = npw // _C
    w0 = _wid() * npw
    pltpu.sync_copy(off_hbm, off_v)

    def drain(s, cnt):
        def w(j, c):
            pltpu.make_async_copy(wv.at[pl.ds(0, 128)], gv.at[0, 0, 0], s).wait()
            return c
        lax.fori_loop(0, cnt, w, 0)

    def load_idx3_fire(ci, par):
        base = pl.multiple_of(w0 + ci * _C, _C)
        pltpu.sync_copy(p_hbm.at[pl.ds(base, _C)], p_v.at[par])
        pltpu.sync_copy(v_hbm.at[pl.ds(base, _C)], v_v.at[par])
        pltpu.sync_copy(ppr_hbm.at[pl.ds(base, _C)], pp_v.at[par])
        off3 = off_v[...]

        def body(k, c):
            sub = k >> 3
            col = (k & 7) * 16
            s16 = pl.ds(k * 16, 16)
            u0 = p_v[par, s16] * jnp.float32(_NB3)
            u1 = v_v[par, s16] * jnp.float32(_NB3)
            u2 = pp_v[par, s16] * jnp.float32(_NB3)
            for t in range(_NT):
                sh = jnp.float32(t / _NT)
                i0 = jnp.minimum((u0 + sh).astype(jnp.int32), _NB3 - 1)
                i1 = jnp.minimum((u1 + sh).astype(jnp.int32), _NB3 - 1)
                i2 = jnp.minimum((u2 + sh).astype(jnp.int32), _NB3 - 1)
                idx3[par, t, sub, pl.ds(col, 16)] = (
                    i0 + i1 * _NB3 + i2 * (_NB3 * _NB3) + (off3 + t * _TBL3P))
            return c
        lax.fori_loop(0, _C // 16, body, 0)

        def f(j, c):
            pltpu.async_copy(wv.at[idx3.at[par, j // _SUB, j % _SUB]],
                             gv.at[par, j // _SUB, j % _SUB], semv)
            return c
        lax.fori_loop(0, _NT * _SUB, f, 0)

    load_idx3_fire(0, 0)

    def chunk(ci, carry):
        par = ci & 1
        drain(semv, _NT * _SUB)

        @pl.when(ci + 1 < nchunks)
        def _():
            load_idx3_fire(ci + 1, 1 - par)

        @pl.when(ci > 1)
        def _():
            pltpu.make_async_copy(p_hbm.at[pl.ds(0, _C)], vp_v.at[0], semo).wait()

        base = pl.multiple_of(w0 + ci * _C, _C)
        _red16(gv.at[par], v_v.at[par], vp_v.at[par], True)
        pltpu.async_copy(vp_v.at[par], out.at[pl.ds(base, _C)], semo)
        return carry

    lax.fori_loop(0, nchunks, chunk, 0)
    for _ in range(2):
        pltpu.make_async_copy(p_hbm.at[pl.ds(0, _C)], vp_v.at[0], semo).wait()


def kernel(state, W_p, W_v, W_r, action):
    n = state.shape[0]
    p_in = state[:, 0]
    v_in = state[:, 1]

    def _phys(W):
        na, nt, nf = W.shape
        return W.reshape(na, nt // 8, 8, nf // 128, 128).transpose(0, 1, 3, 2, 4).reshape(-1)

    wp = _phys(W_p)
    wv = jnp.pad(W_v, ((0, 0), (0, 0), (0, _TBL3P - W_v.shape[2]))).reshape(-1)
    wr = _phys(W_r)
    a = jnp.clip(jnp.asarray(action, jnp.int32), 0, W_p.shape[0] - 1)
    off2 = jnp.full((16,), a * (_NT * _TBL2), dtype=jnp.int32)
    off3 = jnp.full((16,), a * (_NT * _TBL3P), dtype=jnp.int32)

    mesh = plsc.VectorSubcoreMesh(core_axis_name="c", subcore_axis_name="s")
    ppr = pl.kernel(
        functools.partial(_k1_body, n=n),
        out_type=jax.ShapeDtypeStruct((2 * n,), jnp.float32),
        mesh=mesh,
        scratch_types=[
            pltpu.VMEM((2, _C), jnp.float32),
            pltpu.VMEM((2, _C), jnp.float32),
            pltpu.VMEM((2, _C), jnp.float32),
            pltpu.VMEM((2, _C), jnp.float32),
            pltpu.VMEM((2, _NT, _SUB, 128), jnp.int32),
            pltpu.VMEM((2, _NT, _SUB, 128), jnp.float32),
            pltpu.VMEM((2, _NT, _SUB, 128), jnp.float32),
            pltpu.VMEM((16,), jnp.int32),
            pltpu.SemaphoreType.DMA,
            pltpu.SemaphoreType.DMA,
            pltpu.SemaphoreType.DMA,
        ],
    )(p_in, v_in, wp, wr, off2)

    vpr = pl.kernel(
        functools.partial(_k2_body, n=n),
        out_type=jax.ShapeDtypeStruct((n,), jnp.float32),
        mesh=mesh,
        scratch_types=[
            pltpu.VMEM((2, _C), jnp.float32),
            pltpu.VMEM((2, _C), jnp.float32),
            pltpu.VMEM((2, _C), jnp.float32),
            pltpu.VMEM((2, _C), jnp.float32),
            pltpu.VMEM((2, _NT, _SUB, 128), jnp.int32),
            pltpu.VMEM((2, _NT, _SUB, 128), jnp.float32),
            pltpu.VMEM((16,), jnp.int32),
            pltpu.SemaphoreType.DMA,
            pltpu.SemaphoreType.DMA,
        ],
    )(p_in, v_in, ppr, wv, off3)

    return jnp.stack([ppr[:n], vpr, ppr[n:]], axis=1)
